# baseline (speedup 1.0000x reference)
import jax
import jax.numpy as jnp
from jax.experimental import pallas as pl

_BLK = 2000


def _mlp_body(x_ref, w1_ref, b1_ref, w2_ref, b2_ref, o_ref):
    x = x_ref[...]
    h = jnp.dot(x, w1_ref[...], preferred_element_type=jnp.float32)
    h = jnp.maximum(h + b1_ref[...], 0.0)
    y = jnp.sum(h * w2_ref[...], axis=1, keepdims=True)
    o_ref[...] = y + b2_ref[...]


def kernel(X, W1, b1, W2, b2):
    n, k = X.shape
    d = W1.shape[1]
    blk = _BLK if n % _BLK == 0 else 8
    pad = (-n) % blk
    if pad:
        X = jnp.pad(X, ((0, pad), (0, 0)))
    npad = n + pad

    b1r = b1.reshape(1, d)
    w2r = W2.reshape(1, d)
    b2r = b2.reshape(1, 1)

    out = pl.pallas_call(
        _mlp_body,
        grid=(npad // blk,),
        in_specs=[
            pl.BlockSpec((blk, k), lambda i: (i, 0)),
            pl.BlockSpec((k, d), lambda i: (0, 0)),
            pl.BlockSpec((1, d), lambda i: (0, 0)),
            pl.BlockSpec((1, d), lambda i: (0, 0)),
            pl.BlockSpec((1, 1), lambda i: (0, 0)),
        ],
        out_specs=pl.BlockSpec((blk, 1), lambda i: (i, 0)),
        out_shape=jax.ShapeDtypeStruct((npad, 1), jnp.float32),
    )(X, W1, b1r, w2r, b2r)
    return out[:n] if pad else out


# speedup vs baseline: 1.3697x; 1.3697x over previous
import jax
import jax.numpy as jnp
from jax.experimental import pallas as pl

_BLK = 10000


def _mlp_body(x_ref, w1_ref, b1_ref, w2_ref, b2_ref, o_ref):
    x = x_ref[...]
    h = jnp.dot(x, w1_ref[...], preferred_element_type=jnp.float32)
    h = jnp.maximum(h + b1_ref[...], 0.0)
    y = jnp.dot(h, w2_ref[...], preferred_element_type=jnp.float32)
    o_ref[...] = y + b2_ref[...]


def kernel(X, W1, b1, W2, b2):
    n, k = X.shape
    d = W1.shape[1]
    blk = _BLK if n % _BLK == 0 else 8
    pad = (-n) % blk
    if pad:
        X = jnp.pad(X, ((0, pad), (0, 0)))
    npad = n + pad

    b1r = b1.reshape(1, d)
    b2r = b2.reshape(1, 1)

    out = pl.pallas_call(
        _mlp_body,
        grid=(npad // blk,),
        in_specs=[
            pl.BlockSpec((blk, k), lambda i: (i, 0)),
            pl.BlockSpec((k, d), lambda i: (0, 0)),
            pl.BlockSpec((1, d), lambda i: (0, 0)),
            pl.BlockSpec((d, 1), lambda i: (0, 0)),
            pl.BlockSpec((1, 1), lambda i: (0, 0)),
        ],
        out_specs=pl.BlockSpec((blk, 1), lambda i: (i, 0)),
        out_shape=jax.ShapeDtypeStruct((npad, 1), jnp.float32),
    )(X, W1, b1r, W2, b2r)
    return out[:n] if pad else out


# speedup vs baseline: 2.9126x; 2.1265x over previous
import jax
import jax.numpy as jnp
from jax.experimental import pallas as pl

_BLK = 10000


def _mlp_body(x_ref, w1_ref, b1_ref, w2_ref, b2_ref, o_ref):
    x = x_ref[...]
    h = jnp.dot(x, w1_ref[...], preferred_element_type=jnp.float32)
    h = jnp.maximum(h + b1_ref[...], 0.0)
    y = jnp.dot(h, w2_ref[...], preferred_element_type=jnp.float32)
    o_ref[...] = jnp.transpose(y + b2_ref[...], (1, 0)).reshape(o_ref.shape)


def kernel(X, W1, b1, W2, b2):
    n, k = X.shape
    d = W1.shape[1]
    blk = _BLK if n % _BLK == 0 else 8
    pad = (-n) % blk
    if pad:
        X = jnp.pad(X, ((0, pad), (0, 0)))
    npad = n + pad

    b1r = b1.reshape(1, d)
    b2r = b2.reshape(1, 1)

    out = pl.pallas_call(
        _mlp_body,
        grid=(npad // blk,),
        in_specs=[
            pl.BlockSpec((blk, k), lambda i: (i, 0)),
            pl.BlockSpec((k, d), lambda i: (0, 0)),
            pl.BlockSpec((1, d), lambda i: (0, 0)),
            pl.BlockSpec((d, 1), lambda i: (0, 0)),
            pl.BlockSpec((1, 1), lambda i: (0, 0)),
        ],
        out_specs=pl.BlockSpec((1, 1, blk), lambda i: (i, 0, 0)),
        out_shape=jax.ShapeDtypeStruct((npad // blk, 1, blk), jnp.float32),
    )(X, W1, b1r, W2, b2r)
    out = out.reshape(npad, 1)
    return out[:n] if pad else out
